# initial kernel scaffold (unmeasured)
import jax
import jax.numpy as jnp
from jax import lax
from jax.experimental import pallas as pl
from jax.experimental.pallas import tpu as pltpu

P = 32


def kernel(x, w_mat, scale_x, scale_w):
    m, k_sh = x.shape
    k_sh2, n = w_mat.shape
    R = m // P

    def body(x_ref, w_ref, sx_ref, sw_ref, out_ref,
             comm_ref, wbf_ref, send_sems, recv_sems, credit_sem, copy_sem):
        my = lax.axis_index("i")
        left = lax.rem(my + P - 1, P)
        right = lax.rem(my + 1, P)

        barrier_sem = pltpu.get_barrier_semaphore()
        for nbr in (left, right):
            pl.semaphore_signal(barrier_sem, inc=1, device_id=(nbr,),
                                device_id_type=pl.DeviceIdType.MESH)
        pl.semaphore_wait(barrier_sem, 2)

        wbf_ref[...] = w_ref[...].astype(jnp.bfloat16)
        scale = sx_ref[0] * sw_ref[0]

        def partial_chunk(c):
            xs = x_ref[pl.ds(c * R, R), :].astype(jnp.bfloat16)
            return lax.dot_general(xs, wbf_ref[...],
                                   (((1,), (0,)), ((), ())),
                                   preferred_element_type=jnp.float32)

        def ring_send(slot, rslot):
            return pltpu.make_async_remote_copy(
                src_ref=comm_ref.at[slot],
                dst_ref=comm_ref.at[rslot],
                send_sem=send_sems.at[slot],
                recv_sem=recv_sems.at[rslot],
                device_id=(right,),
                device_id_type=pl.DeviceIdType.MESH,
            )

        def credit_to_left():
            pl.semaphore_signal(credit_sem, inc=1, device_id=(left,),
                                device_id_type=pl.DeviceIdType.MESH)

        comm_ref[0] = partial_chunk(my)

        def rs_step(s, carry):
            slot = lax.rem(s, 2)
            rslot = lax.rem(s + 1, 2)

            @pl.when(s >= 1)
            def _():
                pl.semaphore_wait(credit_sem, 1)

            rdma = ring_send(slot, rslot)
            rdma.start()
            rdma.wait()
            credit_to_left()
            c = lax.rem(my - s - 1 + 2 * P, P)
            comm_ref[rslot] = comm_ref[rslot] + partial_chunk(c)
            return carry

        lax.fori_loop(0, P - 1, rs_step, 0)

        own_slot = (P - 1) % 2
        comm_ref[own_slot] = jnp.maximum(comm_ref[own_slot] * scale, 0.0)
        g = lax.rem(my + 1, P)
        cp = pltpu.make_async_copy(comm_ref.at[own_slot],
                                   out_ref.at[pl.ds(g * R, R)], copy_sem)
        cp.start()
        cp.wait()

        def ag_step(t, carry):
            k = t + (P - 1)
            slot = lax.rem(k, 2)
            rslot = lax.rem(k + 1, 2)
            pl.semaphore_wait(credit_sem, 1)
            rdma = ring_send(slot, rslot)
            rdma.start()
            rdma.wait()

            @pl.when(t <= P - 3)
            def _():
                credit_to_left()

            c = lax.rem(my - t + 2 * P, P)
            cp2 = pltpu.make_async_copy(comm_ref.at[rslot],
                                        out_ref.at[pl.ds(c * R, R)], copy_sem)
            cp2.start()
            cp2.wait()
            return carry

        lax.fori_loop(0, P - 1, ag_step, 0)

    out_shape = jax.ShapeDtypeStruct((m, n), jnp.float32)
    return pl.pallas_call(
        body,
        out_shape=out_shape,
        in_specs=[
            pl.BlockSpec(memory_space=pltpu.VMEM),
            pl.BlockSpec(memory_space=pltpu.VMEM),
            pl.BlockSpec(memory_space=pltpu.SMEM),
            pl.BlockSpec(memory_space=pltpu.SMEM),
        ],
        out_specs=pl.BlockSpec(memory_space=pltpu.ANY),
        scratch_shapes=[
            pltpu.VMEM((2, R, n), jnp.float32),
            pltpu.VMEM((k_sh2, n), jnp.bfloat16),
            pltpu.SemaphoreType.DMA((2,)),
            pltpu.SemaphoreType.DMA((2,)),
            pltpu.SemaphoreType.REGULAR,
            pltpu.SemaphoreType.DMA,
        ],
        compiler_params=pltpu.CompilerParams(collective_id=0),
    )(x, w_mat, scale_x, scale_w)


# baseline (device time: 3320041 ns/iter reference)
import jax
import jax.numpy as jnp
from jax import lax
from jax.experimental import pallas as pl
from jax.experimental.pallas import tpu as pltpu

P = 32


def kernel(x, w_mat, scale_x, scale_w):
    m, k_sh = x.shape
    k_sh2, n = w_mat.shape
    R = m // P

    def body(x_ref, w_ref, sx_ref, sw_ref, out_ref,
             comm_ref, wbf_ref, send_sems, recv_sems, credit_sem, copy_sem):
        my = lax.axis_index("i")
        left = lax.rem(my + P - 1, P)
        right = lax.rem(my + 1, P)

        barrier_sem = pltpu.get_barrier_semaphore()
        for nbr in (left, right):
            pl.semaphore_signal(barrier_sem, inc=1, device_id=(nbr,),
                                device_id_type=pl.DeviceIdType.MESH)
        pl.semaphore_wait(barrier_sem, 2)

        wbf_ref[...] = w_ref[...].astype(jnp.bfloat16)
        scale = sx_ref[0] * sw_ref[0]

        def partial_chunk(c):
            xs = x_ref[pl.ds(c * R, R), :].astype(jnp.bfloat16)
            return lax.dot_general(xs, wbf_ref[...],
                                   (((1,), (0,)), ((), ())),
                                   preferred_element_type=jnp.float32)

        def ring_send(slot, rslot):
            return pltpu.make_async_remote_copy(
                src_ref=comm_ref.at[slot],
                dst_ref=comm_ref.at[rslot],
                send_sem=send_sems.at[slot],
                recv_sem=recv_sems.at[rslot],
                device_id=(right,),
                device_id_type=pl.DeviceIdType.MESH,
            )

        def credit_to_left():
            pl.semaphore_signal(credit_sem, inc=1, device_id=(left,),
                                device_id_type=pl.DeviceIdType.MESH)

        comm_ref[0] = partial_chunk(my)

        def rs_step(s, carry):
            slot = lax.rem(s, 2)
            rslot = lax.rem(s + 1, 2)

            @pl.when(s >= 1)
            def _():
                pl.semaphore_wait(credit_sem, 1)

            rdma = ring_send(slot, rslot)
            rdma.start()
            rdma.wait()
            credit_to_left()
            c = lax.rem(my - s - 1 + 2 * P, P)
            comm_ref[rslot] = comm_ref[rslot] + partial_chunk(c)
            return carry

        lax.fori_loop(0, P - 1, rs_step, 0)

        own_slot = (P - 1) % 2
        comm_ref[own_slot] = jnp.maximum(comm_ref[own_slot] * scale, 0.0)
        g = lax.rem(my + 1, P)
        cp = pltpu.make_async_copy(comm_ref.at[own_slot],
                                   out_ref.at[pl.ds(g * R, R)], copy_sem)
        cp.start()
        cp.wait()

        def ag_step(t, carry):
            k = t + (P - 1)
            slot = lax.rem(k, 2)
            rslot = lax.rem(k + 1, 2)
            pl.semaphore_wait(credit_sem, 1)
            rdma = ring_send(slot, rslot)
            rdma.start()
            rdma.wait()

            @pl.when(t <= P - 3)
            def _():
                credit_to_left()

            c = lax.rem(my - t + 2 * P, P)
            cp2 = pltpu.make_async_copy(comm_ref.at[rslot],
                                        out_ref.at[pl.ds(c * R, R)], copy_sem)
            cp2.start()
            cp2.wait()
            return carry

        lax.fori_loop(0, P - 1, ag_step, 0)

    out_shape = jax.ShapeDtypeStruct((m, n), jnp.float32)
    return pl.pallas_call(
        body,
        out_shape=out_shape,
        in_specs=[
            pl.BlockSpec(memory_space=pltpu.VMEM),
            pl.BlockSpec(memory_space=pltpu.VMEM),
            pl.BlockSpec(memory_space=pltpu.SMEM),
            pl.BlockSpec(memory_space=pltpu.SMEM),
        ],
        out_specs=pl.BlockSpec(memory_space=pl.ANY),
        scratch_shapes=[
            pltpu.VMEM((2, R, n), jnp.float32),
            pltpu.VMEM((k_sh2, n), jnp.bfloat16),
            pltpu.SemaphoreType.DMA((2,)),
            pltpu.SemaphoreType.DMA((2,)),
            pltpu.SemaphoreType.REGULAR,
            pltpu.SemaphoreType.DMA,
        ],
        compiler_params=pltpu.CompilerParams(collective_id=0),
    )(x, w_mat, scale_x, scale_w)


# device time: 1711716 ns/iter; 1.9396x vs baseline; 1.9396x over previous
import jax
import jax.numpy as jnp
from jax import lax
from jax.experimental import pallas as pl
from jax.experimental.pallas import tpu as pltpu

P = 32

RING_ORDER = [
    0, 8, 16, 24, 25, 17, 9, 1,
    2, 10, 18, 26, 29, 21, 13, 5,
    6, 14, 22, 30, 31, 23, 15, 7,
    4, 12, 20, 28, 27, 19, 11, 3,
]
INV_ORDER = [0] * P
for _p, _l in enumerate(RING_ORDER):
    INV_ORDER[_l] = _p


def kernel(x, w_mat, scale_x, scale_w):
    m, k_sh = x.shape
    k_sh2, n = w_mat.shape
    R = m // P
    hn = n // 2

    ring = jnp.asarray(RING_ORDER, jnp.int32)
    inv = jnp.asarray(INV_ORDER, jnp.int32)
    my = lax.axis_index("i")
    pos = inv[my]
    nxt = ring[(pos + 1) % P]
    prv = ring[(pos + P - 1) % P]
    meta = jnp.stack([pos, nxt, prv]).astype(jnp.int32)

    def body(meta_ref, x_ref, w_ref, sx_ref, sw_ref, out_ref,
             cw_ref, ccw_ref, wbf_ref,
             send_cw, recv_cw, send_ccw, recv_ccw,
             credit_cw, credit_ccw, copy_sem):
        pos = meta_ref[0]
        nxt = meta_ref[1]
        prv = meta_ref[2]

        barrier_sem = pltpu.get_barrier_semaphore()
        for nbr in (prv, nxt):
            pl.semaphore_signal(barrier_sem, inc=1, device_id=(nbr,),
                                device_id_type=pl.DeviceIdType.MESH)
        pl.semaphore_wait(barrier_sem, 2)

        wbf_ref[...] = w_ref[...].astype(jnp.bfloat16)
        scale = sx_ref[0] * sw_ref[0]

        def partial_half(c, half):
            xs = x_ref[pl.ds(c * R, R), :].astype(jnp.bfloat16)
            return lax.dot_general(
                xs, wbf_ref[:, half * hn:(half + 1) * hn],
                (((1,), (0,)), ((), ())),
                preferred_element_type=jnp.float32)

        def ring_send(buf, sends, recvs, slot, rslot, dst):
            return pltpu.make_async_remote_copy(
                src_ref=buf.at[slot],
                dst_ref=buf.at[rslot],
                send_sem=sends.at[slot],
                recv_sem=recvs.at[rslot],
                device_id=(dst,),
                device_id_type=pl.DeviceIdType.MESH,
            )

        def credit(sem, dst):
            pl.semaphore_signal(sem, inc=1, device_id=(dst,),
                                device_id_type=pl.DeviceIdType.MESH)

        def store_out(buf, slot, c, half):
            cp = pltpu.make_async_copy(
                buf.at[slot],
                out_ref.at[pl.ds(c * R, R), pl.ds(half * hn, hn)],
                copy_sem)
            cp.start()
            cp.wait()

        cw_ref[0] = partial_half(pos, 0)
        ccw_ref[0] = partial_half(lax.rem(pos, P), 1)

        def rs_step(s, carry):
            slot = lax.rem(s, 2)
            rslot = lax.rem(s + 1, 2)

            @pl.when(s >= 1)
            def _():
                pl.semaphore_wait(credit_cw, 1)
                pl.semaphore_wait(credit_ccw, 1)

            r_cw = ring_send(cw_ref, send_cw, recv_cw, slot, rslot, nxt)
            r_ccw = ring_send(ccw_ref, send_ccw, recv_ccw, slot, rslot, prv)
            r_cw.start()
            r_ccw.start()
            r_cw.wait()
            r_ccw.wait()
            credit(credit_cw, prv)
            credit(credit_ccw, nxt)
            c_cw = lax.rem(pos - s - 1 + 2 * P, P)
            c_ccw = lax.rem(pos + s + 1, P)
            cw_ref[rslot] = cw_ref[rslot] + partial_half(c_cw, 0)
            ccw_ref[rslot] = ccw_ref[rslot] + partial_half(c_ccw, 1)
            return carry

        lax.fori_loop(0, P - 1, rs_step, 0)

        own_slot = (P - 1) % 2
        cw_ref[own_slot] = jnp.maximum(cw_ref[own_slot] * scale, 0.0)
        ccw_ref[own_slot] = jnp.maximum(ccw_ref[own_slot] * scale, 0.0)
        g_cw = lax.rem(pos + 1, P)
        g_ccw = lax.rem(pos + P - 1, P)
        store_out(cw_ref, own_slot, g_cw, 0)
        store_out(ccw_ref, own_slot, g_ccw, 1)

        def ag_step(t, carry):
            k = t + (P - 1)
            slot = lax.rem(k, 2)
            rslot = lax.rem(k + 1, 2)
            pl.semaphore_wait(credit_cw, 1)
            pl.semaphore_wait(credit_ccw, 1)
            r_cw = ring_send(cw_ref, send_cw, recv_cw, slot, rslot, nxt)
            r_ccw = ring_send(ccw_ref, send_ccw, recv_ccw, slot, rslot, prv)
            r_cw.start()
            r_ccw.start()
            r_cw.wait()
            r_ccw.wait()

            @pl.when(t <= P - 3)
            def _():
                credit(credit_cw, prv)
                credit(credit_ccw, nxt)

            c_cw = lax.rem(pos - t + 2 * P, P)
            c_ccw = lax.rem(pos + t, P)
            store_out(cw_ref, rslot, c_cw, 0)
            store_out(ccw_ref, rslot, c_ccw, 1)
            return carry

        lax.fori_loop(0, P - 1, ag_step, 0)

    out_shape = jax.ShapeDtypeStruct((m, n), jnp.float32)
    return pl.pallas_call(
        body,
        out_shape=out_shape,
        in_specs=[
            pl.BlockSpec(memory_space=pltpu.SMEM),
            pl.BlockSpec(memory_space=pltpu.VMEM),
            pl.BlockSpec(memory_space=pltpu.VMEM),
            pl.BlockSpec(memory_space=pltpu.SMEM),
            pl.BlockSpec(memory_space=pltpu.SMEM),
        ],
        out_specs=pl.BlockSpec(memory_space=pl.ANY),
        scratch_shapes=[
            pltpu.VMEM((2, R, hn), jnp.float32),
            pltpu.VMEM((2, R, hn), jnp.float32),
            pltpu.VMEM((k_sh2, n), jnp.bfloat16),
            pltpu.SemaphoreType.DMA((2,)),
            pltpu.SemaphoreType.DMA((2,)),
            pltpu.SemaphoreType.DMA((2,)),
            pltpu.SemaphoreType.DMA((2,)),
            pltpu.SemaphoreType.REGULAR,
            pltpu.SemaphoreType.REGULAR,
            pltpu.SemaphoreType.DMA,
        ],
        compiler_params=pltpu.CompilerParams(collective_id=0),
    )(meta, x, w_mat, scale_x, scale_w)


# device time: 1020869 ns/iter; 3.2522x vs baseline; 1.6767x over previous
import jax
import jax.numpy as jnp
from jax import lax
from jax.experimental import pallas as pl
from jax.experimental.pallas import tpu as pltpu

P = 32

RING_ORDER = [
    0, 8, 16, 24, 25, 17, 9, 1,
    2, 10, 18, 26, 29, 21, 13, 5,
    6, 14, 22, 30, 31, 23, 15, 7,
    4, 12, 20, 28, 27, 19, 11, 3,
]
INV_ORDER = [0] * P
for _p, _l in enumerate(RING_ORDER):
    INV_ORDER[_l] = _p


def kernel(x, w_mat, scale_x, scale_w):
    m, k_sh = x.shape
    k_sh2, n = w_mat.shape
    R = m // P
    hn = n // 2

    ring = jnp.asarray(RING_ORDER, jnp.int32)
    inv = jnp.asarray(INV_ORDER, jnp.int32)
    my = lax.axis_index("i")
    pos = inv[my]
    nxt = ring[(pos + 1) % P]
    prv = ring[(pos + P - 1) % P]
    meta = jnp.stack([pos, nxt, prv]).astype(jnp.int32)

    def body(meta_ref, x_ref, w_ref, sx_ref, sw_ref, out_ref,
             cw_ref, ccw_ref, wbf_ref, stage_cw, stage_ccw,
             send_cw, recv_cw, send_ccw, recv_ccw,
             credit_cw, credit_ccw, copy_sem):
        pos = meta_ref[0]
        nxt = meta_ref[1]
        prv = meta_ref[2]

        barrier_sem = pltpu.get_barrier_semaphore()
        for nbr in (prv, nxt):
            pl.semaphore_signal(barrier_sem, inc=1, device_id=(nbr,),
                                device_id_type=pl.DeviceIdType.MESH)
        pl.semaphore_wait(barrier_sem, 2)

        wbf_ref[...] = w_ref[...].astype(jnp.bfloat16)
        scale = sx_ref[0] * sw_ref[0]

        def partial_half(c, half):
            xs = x_ref[pl.ds(c * R, R), :].astype(jnp.bfloat16)
            return lax.dot_general(
                xs, wbf_ref[:, half * hn:(half + 1) * hn],
                (((1,), (0,)), ((), ())),
                preferred_element_type=jnp.float32)

        def ring_send(buf, sends, recvs, slot, rslot, dst):
            return pltpu.make_async_remote_copy(
                src_ref=buf.at[slot],
                dst_ref=buf.at[rslot],
                send_sem=sends.at[slot],
                recv_sem=recvs.at[rslot],
                device_id=(dst,),
                device_id_type=pl.DeviceIdType.MESH,
            )

        def credit(sem, dst):
            pl.semaphore_signal(sem, inc=1, device_id=(dst,),
                                device_id_type=pl.DeviceIdType.MESH)

        def store_out(stage, c, half):
            cp = pltpu.make_async_copy(
                stage,
                out_ref.at[pl.ds(c * R, R), pl.ds(half * hn, hn)],
                copy_sem)
            cp.start()
            cp.wait()

        cw_ref[0] = partial_half(pos, 0).astype(jnp.bfloat16)
        ccw_ref[0] = partial_half(lax.rem(pos, P), 1).astype(jnp.bfloat16)

        def rs_step(s, carry):
            slot = lax.rem(s, 2)
            rslot = lax.rem(s + 1, 2)

            @pl.when(s >= 1)
            def _():
                pl.semaphore_wait(credit_cw, 1)
                pl.semaphore_wait(credit_ccw, 1)

            r_cw = ring_send(cw_ref, send_cw, recv_cw, slot, rslot, nxt)
            r_ccw = ring_send(ccw_ref, send_ccw, recv_ccw, slot, rslot, prv)
            r_cw.start()
            r_ccw.start()
            r_cw.wait()
            r_ccw.wait()
            credit(credit_cw, prv)
            credit(credit_ccw, nxt)
            c_cw = lax.rem(pos - s - 1 + 2 * P, P)
            c_ccw = lax.rem(pos + s + 1, P)
            cw_ref[rslot] = (cw_ref[rslot].astype(jnp.float32)
                             + partial_half(c_cw, 0)).astype(jnp.bfloat16)
            ccw_ref[rslot] = (ccw_ref[rslot].astype(jnp.float32)
                              + partial_half(c_ccw, 1)).astype(jnp.bfloat16)
            return carry

        lax.fori_loop(0, P - 1, rs_step, 0)

        own_slot = (P - 1) % 2
        y_cw = jnp.maximum(cw_ref[own_slot].astype(jnp.float32) * scale, 0.0)
        y_ccw = jnp.maximum(ccw_ref[own_slot].astype(jnp.float32) * scale, 0.0)
        stage_cw[...] = y_cw
        stage_ccw[...] = y_ccw
        cw_ref[own_slot] = y_cw.astype(jnp.bfloat16)
        ccw_ref[own_slot] = y_ccw.astype(jnp.bfloat16)
        g_cw = lax.rem(pos + 1, P)
        g_ccw = lax.rem(pos + P - 1, P)
        store_out(stage_cw, g_cw, 0)
        store_out(stage_ccw, g_ccw, 1)

        def ag_step(t, carry):
            k = t + (P - 1)
            slot = lax.rem(k, 2)
            rslot = lax.rem(k + 1, 2)
            pl.semaphore_wait(credit_cw, 1)
            pl.semaphore_wait(credit_ccw, 1)
            r_cw = ring_send(cw_ref, send_cw, recv_cw, slot, rslot, nxt)
            r_ccw = ring_send(ccw_ref, send_ccw, recv_ccw, slot, rslot, prv)
            r_cw.start()
            r_ccw.start()
            r_cw.wait()
            r_ccw.wait()

            @pl.when(t <= P - 3)
            def _():
                credit(credit_cw, prv)
                credit(credit_ccw, nxt)

            c_cw = lax.rem(pos - t + 2 * P, P)
            c_ccw = lax.rem(pos + t, P)
            stage_cw[...] = cw_ref[rslot].astype(jnp.float32)
            stage_ccw[...] = ccw_ref[rslot].astype(jnp.float32)
            store_out(stage_cw, c_cw, 0)
            store_out(stage_ccw, c_ccw, 1)
            return carry

        lax.fori_loop(0, P - 1, ag_step, 0)

    out_shape = jax.ShapeDtypeStruct((m, n), jnp.float32)
    return pl.pallas_call(
        body,
        out_shape=out_shape,
        in_specs=[
            pl.BlockSpec(memory_space=pltpu.SMEM),
            pl.BlockSpec(memory_space=pltpu.VMEM),
            pl.BlockSpec(memory_space=pltpu.VMEM),
            pl.BlockSpec(memory_space=pltpu.SMEM),
            pl.BlockSpec(memory_space=pltpu.SMEM),
        ],
        out_specs=pl.BlockSpec(memory_space=pl.ANY),
        scratch_shapes=[
            pltpu.VMEM((2, R, hn), jnp.bfloat16),
            pltpu.VMEM((2, R, hn), jnp.bfloat16),
            pltpu.VMEM((k_sh2, n), jnp.bfloat16),
            pltpu.VMEM((R, hn), jnp.float32),
            pltpu.VMEM((R, hn), jnp.float32),
            pltpu.SemaphoreType.DMA((2,)),
            pltpu.SemaphoreType.DMA((2,)),
            pltpu.SemaphoreType.DMA((2,)),
            pltpu.SemaphoreType.DMA((2,)),
            pltpu.SemaphoreType.REGULAR,
            pltpu.SemaphoreType.REGULAR,
            pltpu.SemaphoreType.DMA,
        ],
        compiler_params=pltpu.CompilerParams(collective_id=0),
    )(meta, x, w_mat, scale_x, scale_w)


# device time: 952507 ns/iter; 3.4856x vs baseline; 1.0718x over previous
import jax
import jax.numpy as jnp
from jax import lax
from jax.experimental import pallas as pl
from jax.experimental.pallas import tpu as pltpu

P = 32

RING_ORDER = [
    0, 8, 16, 24, 25, 17, 9, 1,
    2, 10, 18, 26, 29, 21, 13, 5,
    6, 14, 22, 30, 31, 23, 15, 7,
    4, 12, 20, 28, 27, 19, 11, 3,
]
INV_ORDER = [0] * P
for _p, _l in enumerate(RING_ORDER):
    INV_ORDER[_l] = _p


def kernel(x, w_mat, scale_x, scale_w):
    m, k_sh = x.shape
    k_sh2, n = w_mat.shape
    R = m // P
    hn = n // 2

    ring = jnp.asarray(RING_ORDER, jnp.int32)
    inv = jnp.asarray(INV_ORDER, jnp.int32)
    my = lax.axis_index("i")
    pos = inv[my]
    nxt = ring[(pos + 1) % P]
    prv = ring[(pos + P - 1) % P]
    meta = jnp.stack([pos, nxt, prv]).astype(jnp.int32)

    def body(meta_ref, x_ref, w_ref, sx_ref, sw_ref, out_ref,
             cw_ref, ccw_ref, wbf_ref, stage_cw, stage_ccw,
             send_cw, recv_cw, send_ccw, recv_ccw,
             credit_cw, credit_ccw, copy_cw, copy_ccw):
        pos = meta_ref[0]
        nxt = meta_ref[1]
        prv = meta_ref[2]

        barrier_sem = pltpu.get_barrier_semaphore()
        for nbr in (prv, nxt):
            pl.semaphore_signal(barrier_sem, inc=1, device_id=(nbr,),
                                device_id_type=pl.DeviceIdType.MESH)
        pl.semaphore_wait(barrier_sem, 2)

        wbf_ref[...] = w_ref[...].astype(jnp.bfloat16)
        scale = sx_ref[0] * sw_ref[0]

        def partial_half(c, half):
            xs = x_ref[pl.ds(c * R, R), :].astype(jnp.bfloat16)
            return lax.dot_general(
                xs, wbf_ref[:, half * hn:(half + 1) * hn],
                (((1,), (0,)), ((), ())),
                preferred_element_type=jnp.float32)

        def ring_send(buf, sends, recvs, slot, rslot, dst):
            return pltpu.make_async_remote_copy(
                src_ref=buf.at[slot],
                dst_ref=buf.at[rslot],
                send_sem=sends.at[slot],
                recv_sem=recvs.at[rslot],
                device_id=(dst,),
                device_id_type=pl.DeviceIdType.MESH,
            )

        def credit(sem, dst):
            pl.semaphore_signal(sem, inc=1, device_id=(dst,),
                                device_id_type=pl.DeviceIdType.MESH)

        def out_copy(stage, slot, c, half, sems):
            return pltpu.make_async_copy(
                stage.at[slot],
                out_ref.at[pl.ds(c * R, R), pl.ds(half * hn, hn)],
                sems.at[slot])

        cw_ref[0] = partial_half(pos, 0).astype(jnp.bfloat16)
        ccw_ref[0] = partial_half(lax.rem(pos, P), 1).astype(jnp.bfloat16)

        def rs_step(s, carry):
            slot = lax.rem(s, 2)
            rslot = lax.rem(s + 1, 2)

            @pl.when(s >= 1)
            def _():
                pl.semaphore_wait(credit_cw, 1)
                pl.semaphore_wait(credit_ccw, 1)

            r_cw = ring_send(cw_ref, send_cw, recv_cw, slot, rslot, nxt)
            r_ccw = ring_send(ccw_ref, send_ccw, recv_ccw, slot, rslot, prv)
            r_cw.start()
            r_ccw.start()
            c_cw = lax.rem(pos - s - 1 + 2 * P, P)
            c_ccw = lax.rem(pos + s + 1, P)
            pc_cw = partial_half(c_cw, 0)
            pc_ccw = partial_half(c_ccw, 1)
            r_cw.wait()
            r_ccw.wait()
            credit(credit_cw, prv)
            credit(credit_ccw, nxt)
            cw_ref[rslot] = (cw_ref[rslot].astype(jnp.float32)
                             + pc_cw).astype(jnp.bfloat16)
            ccw_ref[rslot] = (ccw_ref[rslot].astype(jnp.float32)
                              + pc_ccw).astype(jnp.bfloat16)
            return carry

        lax.fori_loop(0, P - 1, rs_step, 0)

        own_slot = (P - 1) % 2
        y_cw = jnp.maximum(cw_ref[own_slot].astype(jnp.float32) * scale, 0.0)
        y_ccw = jnp.maximum(ccw_ref[own_slot].astype(jnp.float32) * scale, 0.0)
        stage_cw[1] = y_cw
        stage_ccw[1] = y_ccw
        cw_ref[own_slot] = y_cw.astype(jnp.bfloat16)
        ccw_ref[own_slot] = y_ccw.astype(jnp.bfloat16)
        g_cw = lax.rem(pos + 1, P)
        g_ccw = lax.rem(pos + P - 1, P)
        cp1 = out_copy(stage_cw, 1, g_cw, 0, copy_cw)
        cp2 = out_copy(stage_ccw, 1, g_ccw, 1, copy_ccw)
        cp1.start()
        cp2.start()
        cp1.wait()
        cp2.wait()

        def ag_step(t, carry):
            k = t + (P - 1)
            slot = lax.rem(k, 2)
            rslot = lax.rem(k + 1, 2)
            pl.semaphore_wait(credit_cw, 1)
            pl.semaphore_wait(credit_ccw, 1)
            r_cw = ring_send(cw_ref, send_cw, recv_cw, slot, rslot, nxt)
            r_ccw = ring_send(ccw_ref, send_ccw, recv_ccw, slot, rslot, prv)
            r_cw.start()
            r_ccw.start()
            r_cw.wait()
            r_ccw.wait()

            @pl.when(t <= P - 3)
            def _():
                credit(credit_cw, prv)
                credit(credit_ccw, nxt)

            c_cw = lax.rem(pos - t + 2 * P, P)
            c_ccw = lax.rem(pos + t, P)

            @pl.when(t >= 2)
            def _():
                out_copy(stage_cw, rslot, c_cw, 0, copy_cw).wait()
                out_copy(stage_ccw, rslot, c_ccw, 1, copy_ccw).wait()

            stage_cw[rslot] = cw_ref[rslot].astype(jnp.float32)
            stage_ccw[rslot] = ccw_ref[rslot].astype(jnp.float32)
            out_copy(stage_cw, rslot, c_cw, 0, copy_cw).start()
            out_copy(stage_ccw, rslot, c_ccw, 1, copy_ccw).start()
            return carry

        lax.fori_loop(0, P - 1, ag_step, 0)

        for sl in (0, 1):
            out_copy(stage_cw, sl, lax.rem(pos, P), 0, copy_cw).wait()
            out_copy(stage_ccw, sl, lax.rem(pos, P), 1, copy_ccw).wait()

    out_shape = jax.ShapeDtypeStruct((m, n), jnp.float32)
    return pl.pallas_call(
        body,
        out_shape=out_shape,
        in_specs=[
            pl.BlockSpec(memory_space=pltpu.SMEM),
            pl.BlockSpec(memory_space=pltpu.VMEM),
            pl.BlockSpec(memory_space=pltpu.VMEM),
            pl.BlockSpec(memory_space=pltpu.SMEM),
            pl.BlockSpec(memory_space=pltpu.SMEM),
        ],
        out_specs=pl.BlockSpec(memory_space=pl.ANY),
        scratch_shapes=[
            pltpu.VMEM((2, R, hn), jnp.bfloat16),
            pltpu.VMEM((2, R, hn), jnp.bfloat16),
            pltpu.VMEM((k_sh2, n), jnp.bfloat16),
            pltpu.VMEM((2, R, hn), jnp.float32),
            pltpu.VMEM((2, R, hn), jnp.float32),
            pltpu.SemaphoreType.DMA((2,)),
            pltpu.SemaphoreType.DMA((2,)),
            pltpu.SemaphoreType.DMA((2,)),
            pltpu.SemaphoreType.DMA((2,)),
            pltpu.SemaphoreType.REGULAR,
            pltpu.SemaphoreType.REGULAR,
            pltpu.SemaphoreType.DMA((2,)),
            pltpu.SemaphoreType.DMA((2,)),
        ],
        compiler_params=pltpu.CompilerParams(collective_id=0),
    )(meta, x, w_mat, scale_x, scale_w)


# device time: 952358 ns/iter; 3.4861x vs baseline; 1.0002x over previous
import jax
import jax.numpy as jnp
from jax import lax
from jax.experimental import pallas as pl
from jax.experimental.pallas import tpu as pltpu

P = 32

RING_ORDER = [
    0, 8, 16, 24, 25, 17, 9, 1,
    2, 10, 18, 26, 29, 21, 13, 5,
    6, 14, 22, 30, 31, 23, 15, 7,
    4, 12, 20, 28, 27, 19, 11, 3,
]
INV_ORDER = [0] * P
for _p, _l in enumerate(RING_ORDER):
    INV_ORDER[_l] = _p


def kernel(x, w_mat, scale_x, scale_w):
    m, k_sh = x.shape
    k_sh2, n = w_mat.shape
    R = m // P
    hn = n // 2

    ring = jnp.asarray(RING_ORDER, jnp.int32)
    inv = jnp.asarray(INV_ORDER, jnp.int32)
    my = lax.axis_index("i")
    pos = inv[my]
    nxt = ring[(pos + 1) % P]
    prv = ring[(pos + P - 1) % P]
    meta = jnp.stack([pos, nxt, prv]).astype(jnp.int32)

    def body(meta_ref, x_ref, w_ref, sx_ref, sw_ref, out_ref,
             cw_ref, ccw_ref, wbf_ref, stage_cw, stage_ccw,
             send_cw, recv_cw, send_ccw, recv_ccw,
             credit_cw, credit_ccw, copy_cw, copy_ccw):
        pos = meta_ref[0]
        nxt = meta_ref[1]
        prv = meta_ref[2]

        barrier_sem = pltpu.get_barrier_semaphore()
        for nbr in (prv, nxt):
            pl.semaphore_signal(barrier_sem, inc=1, device_id=(nbr,),
                                device_id_type=pl.DeviceIdType.MESH)
        pl.semaphore_wait(barrier_sem, 2)

        wbf_ref[...] = w_ref[...].astype(jnp.bfloat16)
        scale = sx_ref[0] * sw_ref[0]

        def partial_half(c, half):
            xs = x_ref[pl.ds(c * R, R), :].astype(jnp.bfloat16)
            return lax.dot_general(
                xs, wbf_ref[:, half * hn:(half + 1) * hn],
                (((1,), (0,)), ((), ())),
                preferred_element_type=jnp.float32)

        def ring_send(buf, sends, recvs, slot, rslot, dst):
            return pltpu.make_async_remote_copy(
                src_ref=buf.at[slot],
                dst_ref=buf.at[rslot],
                send_sem=sends.at[slot],
                recv_sem=recvs.at[rslot],
                device_id=(dst,),
                device_id_type=pl.DeviceIdType.MESH,
            )

        def credit(sem, dst):
            pl.semaphore_signal(sem, inc=1, device_id=(dst,),
                                device_id_type=pl.DeviceIdType.MESH)

        def out_copy(stage, slot, c, half, sems):
            return pltpu.make_async_copy(
                stage.at[slot],
                out_ref.at[pl.ds(c * R, R), pl.ds(half * hn, hn)],
                sems.at[slot])

        cw_ref[0] = partial_half(pos, 0).astype(jnp.bfloat16)
        ccw_ref[0] = partial_half(lax.rem(pos, P), 1).astype(jnp.bfloat16)

        def rs_step(s, carry):
            slot = lax.rem(s, 2)
            rslot = lax.rem(s + 1, 2)

            @pl.when(s >= 1)
            def _():
                pl.semaphore_wait(credit_cw, 1)
                pl.semaphore_wait(credit_ccw, 1)

            r_cw = ring_send(cw_ref, send_cw, recv_cw, slot, rslot, nxt)
            r_ccw = ring_send(ccw_ref, send_ccw, recv_ccw, slot, rslot, prv)
            r_cw.start()
            r_ccw.start()
            c_cw = lax.rem(pos - s - 1 + 2 * P, P)
            c_ccw = lax.rem(pos + s + 1, P)
            pc_cw = partial_half(c_cw, 0)
            pc_ccw = partial_half(c_ccw, 1)
            r_cw.wait()
            r_ccw.wait()
            credit(credit_cw, prv)
            credit(credit_ccw, nxt)
            cw_ref[rslot] = cw_ref[rslot] + pc_cw.astype(jnp.bfloat16)
            ccw_ref[rslot] = ccw_ref[rslot] + pc_ccw.astype(jnp.bfloat16)
            return carry

        lax.fori_loop(0, P - 1, rs_step, 0)

        own_slot = (P - 1) % 2
        y_cw = jnp.maximum(cw_ref[own_slot].astype(jnp.float32) * scale, 0.0)
        y_ccw = jnp.maximum(ccw_ref[own_slot].astype(jnp.float32) * scale, 0.0)
        stage_cw[1] = y_cw
        stage_ccw[1] = y_ccw
        cw_ref[own_slot] = y_cw.astype(jnp.bfloat16)
        ccw_ref[own_slot] = y_ccw.astype(jnp.bfloat16)
        g_cw = lax.rem(pos + 1, P)
        g_ccw = lax.rem(pos + P - 1, P)
        cp1 = out_copy(stage_cw, 1, g_cw, 0, copy_cw)
        cp2 = out_copy(stage_ccw, 1, g_ccw, 1, copy_ccw)
        cp1.start()
        cp2.start()
        cp1.wait()
        cp2.wait()

        def ag_step(t, carry):
            k = t + (P - 1)
            slot = lax.rem(k, 2)
            rslot = lax.rem(k + 1, 2)
            pl.semaphore_wait(credit_cw, 1)
            pl.semaphore_wait(credit_ccw, 1)
            r_cw = ring_send(cw_ref, send_cw, recv_cw, slot, rslot, nxt)
            r_ccw = ring_send(ccw_ref, send_ccw, recv_ccw, slot, rslot, prv)
            r_cw.start()
            r_ccw.start()

            @pl.when(t >= 1)
            def _():
                cp_cw = lax.rem(pos - t + 1 + 2 * P, P)
                cp_ccw = lax.rem(pos + t - 1, P)

                @pl.when(t >= 3)
                def _():
                    out_copy(stage_cw, slot, cp_cw, 0, copy_cw).wait()
                    out_copy(stage_ccw, slot, cp_ccw, 1, copy_ccw).wait()

                stage_cw[slot] = cw_ref[slot].astype(jnp.float32)
                stage_ccw[slot] = ccw_ref[slot].astype(jnp.float32)
                out_copy(stage_cw, slot, cp_cw, 0, copy_cw).start()
                out_copy(stage_ccw, slot, cp_ccw, 1, copy_ccw).start()

            r_cw.wait()
            r_ccw.wait()

            @pl.when(t <= P - 3)
            def _():
                credit(credit_cw, prv)
                credit(credit_ccw, nxt)

            return carry

        lax.fori_loop(0, P - 1, ag_step, 0)

        c_fin_cw = lax.rem(pos - (P - 2) + 2 * P, P)
        c_fin_ccw = lax.rem(pos + (P - 2), P)
        out_copy(stage_cw, 0, c_fin_cw, 0, copy_cw).wait()
        out_copy(stage_ccw, 0, c_fin_ccw, 1, copy_ccw).wait()
        stage_cw[0] = cw_ref[0].astype(jnp.float32)
        stage_ccw[0] = ccw_ref[0].astype(jnp.float32)
        f_cw = out_copy(stage_cw, 0, c_fin_cw, 0, copy_cw)
        f_ccw = out_copy(stage_ccw, 0, c_fin_ccw, 1, copy_ccw)
        f_cw.start()
        f_ccw.start()
        f_cw.wait()
        f_ccw.wait()
        out_copy(stage_cw, 1, c_fin_cw, 0, copy_cw).wait()
        out_copy(stage_ccw, 1, c_fin_ccw, 1, copy_ccw).wait()

    out_shape = jax.ShapeDtypeStruct((m, n), jnp.float32)
    return pl.pallas_call(
        body,
        out_shape=out_shape,
        in_specs=[
            pl.BlockSpec(memory_space=pltpu.SMEM),
            pl.BlockSpec(memory_space=pltpu.VMEM),
            pl.BlockSpec(memory_space=pltpu.VMEM),
            pl.BlockSpec(memory_space=pltpu.SMEM),
            pl.BlockSpec(memory_space=pltpu.SMEM),
        ],
        out_specs=pl.BlockSpec(memory_space=pl.ANY),
        scratch_shapes=[
            pltpu.VMEM((2, R, hn), jnp.bfloat16),
            pltpu.VMEM((2, R, hn), jnp.bfloat16),
            pltpu.VMEM((k_sh2, n), jnp.bfloat16),
            pltpu.VMEM((2, R, hn), jnp.float32),
            pltpu.VMEM((2, R, hn), jnp.float32),
            pltpu.SemaphoreType.DMA((2,)),
            pltpu.SemaphoreType.DMA((2,)),
            pltpu.SemaphoreType.DMA((2,)),
            pltpu.SemaphoreType.DMA((2,)),
            pltpu.SemaphoreType.REGULAR,
            pltpu.SemaphoreType.REGULAR,
            pltpu.SemaphoreType.DMA((2,)),
            pltpu.SemaphoreType.DMA((2,)),
        ],
        compiler_params=pltpu.CompilerParams(collective_id=0),
    )(meta, x, w_mat, scale_x, scale_w)


# device time: 806498 ns/iter; 4.1166x vs baseline; 1.1809x over previous
import jax
import jax.numpy as jnp
from jax import lax
from jax.experimental import pallas as pl
from jax.experimental.pallas import tpu as pltpu

P = 32

RING_ORDER = [
    0, 8, 16, 24, 25, 17, 9, 1,
    2, 10, 18, 26, 29, 21, 13, 5,
    6, 14, 22, 30, 31, 23, 15, 7,
    4, 12, 20, 28, 27, 19, 11, 3,
]
INV_ORDER = [0] * P
for _p, _l in enumerate(RING_ORDER):
    INV_ORDER[_l] = _p


def kernel(x, w_mat, scale_x, scale_w):
    m, k_sh = x.shape
    k_sh2, n = w_mat.shape
    R = m // P
    hn = n // 2
    qn = n // 4

    ring = jnp.asarray(RING_ORDER, jnp.int32)
    inv = jnp.asarray(INV_ORDER, jnp.int32)
    my = lax.axis_index("i")
    pos = inv[my]
    nxt = ring[(pos + 1) % P]
    prv = ring[(pos + P - 1) % P]
    meta = jnp.stack([pos, nxt, prv]).astype(jnp.int32)

    def body(meta_ref, x_ref, w_ref, sx_ref, sw_ref, out_ref,
             com0, com1, com2, com3, wbf_ref, stage_cw, stage_ccw,
             ss0, ss1, ss2, ss3, rs0, rs1, rs2, rs3,
             cr0, cr1, cr2, cr3, copy_cw, copy_ccw):
        pos = meta_ref[0]
        nxt = meta_ref[1]
        prv = meta_ref[2]

        barrier_sem = pltpu.get_barrier_semaphore()
        for nbr in (prv, nxt):
            pl.semaphore_signal(barrier_sem, inc=1, device_id=(nbr,),
                                device_id_type=pl.DeviceIdType.MESH)
        pl.semaphore_wait(barrier_sem, 2)

        wbf_ref[...] = w_ref[...].astype(jnp.bfloat16)
        scale = sx_ref[0] * sw_ref[0]

        rings = [
            (com0, ss0, rs0, cr0, nxt, prv, 0),
            (com2, ss2, rs2, cr2, prv, nxt, 2),
            (com1, ss1, rs1, cr1, nxt, prv, 1),
            (com3, ss3, rs3, cr3, prv, nxt, 3),
        ]

        def partial_q(c, qi):
            xs = x_ref[pl.ds(c * R, R), :].astype(jnp.bfloat16)
            return lax.dot_general(
                xs, wbf_ref[:, qi * qn:(qi + 1) * qn],
                (((1,), (0,)), ((), ())),
                preferred_element_type=jnp.float32)

        def rdma(rg, src_slot, dst_slot):
            com, ss, rsm, _, dst, _, _ = rg
            return pltpu.make_async_remote_copy(
                src_ref=com.at[src_slot],
                dst_ref=com.at[dst_slot],
                send_sem=ss.at[src_slot],
                recv_sem=rsm.at[dst_slot],
                device_id=(dst,),
                device_id_type=pl.DeviceIdType.MESH,
            )

        def give_credit(rg):
            _, _, _, cr, _, cdst, _ = rg
            pl.semaphore_signal(cr, inc=1, device_id=(cdst,),
                                device_id_type=pl.DeviceIdType.MESH)

        def take_credit(rg):
            pl.semaphore_wait(rg[3], 1)

        def out_copy(stage, slot, c, half, sems):
            return pltpu.make_async_copy(
                stage.at[slot],
                out_ref.at[pl.ds(c * R, R), pl.ds(half * hn, hn)],
                sems.at[slot])

        def rs_chunks(s):
            return (lax.rem(pos - s - 1 + 2 * P, P),
                    lax.rem(pos + s + 1, P))

        for rg in rings:
            rg[0][0] = partial_q(pos, rg[6]).astype(jnp.bfloat16)
        for rg in rings:
            rdma(rg, 0, 1).start()

        def rs_step(s, carry):
            slot = lax.rem(s, 2)
            rslot = lax.rem(s + 1, 2)
            c_cw, c_ccw = rs_chunks(s)
            pcs = [partial_q(c_cw, 0), partial_q(c_ccw, 2),
                   partial_q(c_cw, 1), partial_q(c_ccw, 3)]
            for rg, pc in zip(rings, pcs):
                rdma(rg, rslot, rslot).wait_recv()
                rg[0][rslot] = rg[0][rslot] + pc.astype(jnp.bfloat16)
                rdma(rg, slot, slot).wait_send()
                give_credit(rg)
                take_credit(rg)
                rdma(rg, rslot, slot).start()
            return carry

        lax.fori_loop(0, P - 2, rs_step, 0)

        c_cw, c_ccw = rs_chunks(P - 2)
        pcs = [partial_q(c_cw, 0), partial_q(c_ccw, 2),
               partial_q(c_cw, 1), partial_q(c_ccw, 3)]
        for rg, pc in zip(rings, pcs):
            rdma(rg, 1, 1).wait_recv()
            rg[0][1] = rg[0][1] + pc.astype(jnp.bfloat16)
            rdma(rg, 0, 0).wait_send()
            give_credit(rg)

        g_cw = lax.rem(pos + 1, P)
        g_ccw = lax.rem(pos + P - 1, P)
        y0 = jnp.maximum(com0[1].astype(jnp.float32) * scale, 0.0)
        y1 = jnp.maximum(com1[1].astype(jnp.float32) * scale, 0.0)
        y2 = jnp.maximum(com2[1].astype(jnp.float32) * scale, 0.0)
        y3 = jnp.maximum(com3[1].astype(jnp.float32) * scale, 0.0)
        stage_cw[1, :, 0:qn] = y0
        stage_cw[1, :, qn:2 * qn] = y1
        stage_ccw[1, :, 0:qn] = y2
        stage_ccw[1, :, qn:2 * qn] = y3
        com0[1] = y0.astype(jnp.bfloat16)
        com1[1] = y1.astype(jnp.bfloat16)
        com2[1] = y2.astype(jnp.bfloat16)
        com3[1] = y3.astype(jnp.bfloat16)
        ep_cw = out_copy(stage_cw, 1, g_cw, 0, copy_cw)
        ep_ccw = out_copy(stage_ccw, 1, g_ccw, 1, copy_ccw)
        ep_cw.start()
        ep_ccw.start()
        ep_cw.wait()
        ep_ccw.wait()

        for rg in rings:
            take_credit(rg)
            rdma(rg, 1, 0).start()

        def ag_step(t, carry):
            k = t + (P - 1)
            slot = lax.rem(k, 2)
            rslot = lax.rem(k + 1, 2)
            for rg in rings:
                rdma(rg, rslot, rslot).wait_recv()
                rdma(rg, slot, slot).wait_send()
                give_credit(rg)
                take_credit(rg)
                rdma(rg, rslot, slot).start()

            c_cw = lax.rem(pos - t + 2 * P, P)
            c_ccw = lax.rem(pos + t, P)

            @pl.when(t >= 2)
            def _():
                out_copy(stage_cw, rslot, c_cw, 0, copy_cw).wait()
                out_copy(stage_ccw, rslot, c_ccw, 1, copy_ccw).wait()

            stage_cw[rslot, :, 0:qn] = com0[rslot].astype(jnp.float32)
            stage_cw[rslot, :, qn:2 * qn] = com1[rslot].astype(jnp.float32)
            stage_ccw[rslot, :, 0:qn] = com2[rslot].astype(jnp.float32)
            stage_ccw[rslot, :, qn:2 * qn] = com3[rslot].astype(jnp.float32)
            out_copy(stage_cw, rslot, c_cw, 0, copy_cw).start()
            out_copy(stage_ccw, rslot, c_ccw, 1, copy_ccw).start()
            return carry

        lax.fori_loop(0, P - 2, ag_step, 0)

        for rg in rings:
            rdma(rg, 0, 0).wait_recv()
            rdma(rg, 1, 1).wait_send()
        c_fin_cw = lax.rem(pos - (P - 2) + 2 * P, P)
        c_fin_ccw = lax.rem(pos + (P - 2), P)
        out_copy(stage_cw, 0, c_fin_cw, 0, copy_cw).wait()
        out_copy(stage_ccw, 0, c_fin_ccw, 1, copy_ccw).wait()
        stage_cw[0, :, 0:qn] = com0[0].astype(jnp.float32)
        stage_cw[0, :, qn:2 * qn] = com1[0].astype(jnp.float32)
        stage_ccw[0, :, 0:qn] = com2[0].astype(jnp.float32)
        stage_ccw[0, :, qn:2 * qn] = com3[0].astype(jnp.float32)
        f_cw = out_copy(stage_cw, 0, c_fin_cw, 0, copy_cw)
        f_ccw = out_copy(stage_ccw, 0, c_fin_ccw, 1, copy_ccw)
        f_cw.start()
        f_ccw.start()
        f_cw.wait()
        f_ccw.wait()
        out_copy(stage_cw, 1, c_fin_cw, 0, copy_cw).wait()
        out_copy(stage_ccw, 1, c_fin_ccw, 1, copy_ccw).wait()

    out_shape = jax.ShapeDtypeStruct((m, n), jnp.float32)
    return pl.pallas_call(
        body,
        out_shape=out_shape,
        in_specs=[
            pl.BlockSpec(memory_space=pltpu.SMEM),
            pl.BlockSpec(memory_space=pltpu.VMEM),
            pl.BlockSpec(memory_space=pltpu.VMEM),
            pl.BlockSpec(memory_space=pltpu.SMEM),
            pl.BlockSpec(memory_space=pltpu.SMEM),
        ],
        out_specs=pl.BlockSpec(memory_space=pl.ANY),
        scratch_shapes=[
            pltpu.VMEM((2, R, qn), jnp.bfloat16),
            pltpu.VMEM((2, R, qn), jnp.bfloat16),
            pltpu.VMEM((2, R, qn), jnp.bfloat16),
            pltpu.VMEM((2, R, qn), jnp.bfloat16),
            pltpu.VMEM((k_sh2, n), jnp.bfloat16),
            pltpu.VMEM((2, R, hn), jnp.float32),
            pltpu.VMEM((2, R, hn), jnp.float32),
            pltpu.SemaphoreType.DMA((2,)),
            pltpu.SemaphoreType.DMA((2,)),
            pltpu.SemaphoreType.DMA((2,)),
            pltpu.SemaphoreType.DMA((2,)),
            pltpu.SemaphoreType.DMA((2,)),
            pltpu.SemaphoreType.DMA((2,)),
            pltpu.SemaphoreType.DMA((2,)),
            pltpu.SemaphoreType.DMA((2,)),
            pltpu.SemaphoreType.REGULAR,
            pltpu.SemaphoreType.REGULAR,
            pltpu.SemaphoreType.REGULAR,
            pltpu.SemaphoreType.REGULAR,
            pltpu.SemaphoreType.DMA((2,)),
            pltpu.SemaphoreType.DMA((2,)),
        ],
        compiler_params=pltpu.CompilerParams(collective_id=0),
    )(meta, x, w_mat, scale_x, scale_w)
